# Initial kernel scaffold; baseline (speedup 1.0000x reference)
import jax
import jax.numpy as jnp
from jax.experimental import pallas as pl


def kernel(x, edge_index, percent, ricci_curvature, W1, b1, W2, b2, W3, b3, Wout, bout, bn1_w, bn1_b, bn2_w, bn2_b):
    raise NotImplementedError("write your pallas kernel here")


# speedup vs baseline: 10.0055x; 10.0055x over previous
import functools

import jax
import jax.numpy as jnp
from jax import lax
from jax.experimental import pallas as pl
from jax.experimental.pallas import tpu as pltpu
from jax.experimental.pallas import tpu_sc as plsc

NC = 2
NS = 16
NW = NC * NS
CH = 80
DEGW = 8
RB = 1280
BN_C = 1.0 / (1.0 + 1e-5) ** 0.5


def _sc_mesh():
    return plsc.VectorSubcoreMesh(core_axis_name="c", subcore_axis_name="s")


def _npad(n):
    return ((n + NS * CH - 1) // (NS * CH)) * (NS * CH)


@functools.lru_cache(maxsize=None)
def _make_deg(n, e):
    ept = e // NW
    npad = _npad(n)
    grows = npad // 128

    @functools.partial(
        pl.kernel,
        out_type=jax.ShapeDtypeStruct((NW, grows, 128), jnp.float32),
        mesh=_sc_mesh(),
        compiler_params=pltpu.CompilerParams(needs_layout_passes=False),
        scratch_types=[
            pltpu.VMEM((ept,), jnp.int32),
            pltpu.VMEM((grows, 128), jnp.float32),
        ],
    )
    def deg_kernel(dst_hbm, out_hbm, idx_v, deg_v):
        cid = lax.axis_index("c")
        sid = lax.axis_index("s")
        wid = sid * NC + cid

        def zero(i, carry):
            for t in range(8):
                deg_v[i, pl.ds(t * 16, 16)] = jnp.zeros((16,), jnp.float32)
            return carry

        lax.fori_loop(0, grows, zero, 0)
        pltpu.sync_copy(dst_hbm.at[pl.ds(pl.multiple_of(wid * ept, 8), ept)], idx_v)
        ones16 = jnp.ones((16,), jnp.float32)

        def body(j, carry):
            for t in range(5):
                idx = idx_v[pl.ds(j * 80 + t * 16, 16)]
                plsc.addupdate_scatter(deg_v, [idx >> 7, idx & 127], ones16)
            return carry

        lax.fori_loop(0, ept // 80, body, 0)
        pltpu.sync_copy(deg_v, out_hbm.at[wid])

    return deg_kernel


@functools.lru_cache(maxsize=None)
def _make_prop(n, e, d):
    ept = e // NW
    nchunk = ept // CH
    npad = _npad(n)
    slab = npad // NS

    @functools.partial(
        pl.kernel,
        out_type=jax.ShapeDtypeStruct((NC, npad, d), jnp.float32),
        mesh=_sc_mesh(),
        scratch_types=[
            pltpu.VMEM((CH,), jnp.int32),
            pltpu.VMEM((CH,), jnp.int32),
            pltpu.VMEM((CH, d), jnp.float32),
            pltpu.VMEM_SHARED((npad, d), jnp.float32),
            pltpu.SemaphoreType.DMA,
        ],
    )
    def prop_kernel(z_hbm, src_hbm, dst_hbm, zeros_hbm, out_hbm,
                    sidx_v, didx_v, row_v, acc, sem):
        cid = lax.axis_index("c")
        sid = lax.axis_index("s")
        wid = sid * NC + cid
        nwb = slab // CH

        pltpu.sync_copy(zeros_hbm, row_v)

        def zinit(k, carry):
            off = pl.multiple_of(sid * slab + k * CH, 8)
            pltpu.sync_copy(row_v, acc.at[pl.ds(off, CH)])
            return carry

        lax.fori_loop(0, nwb, zinit, 0)
        plsc.subcore_barrier()

        def body(j, carry):
            base = pl.multiple_of(wid * ept + j * CH, 8)
            pltpu.sync_copy(src_hbm.at[pl.ds(base, CH)], sidx_v)
            pltpu.sync_copy(dst_hbm.at[pl.ds(base, CH)], didx_v)
            pltpu.async_copy(z_hbm.at[sidx_v], row_v, sem).wait()
            pltpu.sync_copy(row_v, acc.at[didx_v], add=True)
            return carry

        lax.fori_loop(0, nchunk, body, 0)
        plsc.subcore_barrier()

        def wback(k, carry):
            off = pl.multiple_of(sid * slab + k * CH, 8)
            pltpu.sync_copy(acc.at[pl.ds(off, CH)], row_v)
            pltpu.sync_copy(row_v, out_hbm.at[cid, pl.ds(off, CH)])
            return carry

        lax.fori_loop(0, nwb, wback, 0)

    return prop_kernel


def _row_spec(d):
    return pl.BlockSpec((RB, d), lambda i: (i, 0))


def _full_spec(shape):
    nd = len(shape)
    return pl.BlockSpec(shape, lambda i: (0,) * nd)


def _part_spec(d):
    return pl.BlockSpec((NC, RB, d), lambda i: (0, i, 0))


@functools.lru_cache(maxsize=None)
def _make_tc1(n, din, h):
    G = RB // 128

    def body(x_ref, bnw_ref, bnb_ref, degp_ref, w_ref, dinv_ref, z_ref):
        dsum = jnp.sum(degp_ref[...], axis=0, keepdims=True) + 1.0
        io = lax.broadcasted_iota(jnp.int32, (128, 128), 0)
        ic = lax.broadcasted_iota(jnp.int32, (128, 128), 1)
        eye = (io == ic).astype(jnp.float32)
        onesc = jnp.ones((128, 1), jnp.float32)
        cols = [jnp.dot(eye * dsum[:, g * 128:(g + 1) * 128], onesc,
                        preferred_element_type=jnp.float32,
                        precision=lax.Precision.HIGHEST) for g in range(G)]
        deg = jnp.concatenate(cols, axis=0)
        dinv = lax.rsqrt(jnp.maximum(deg, 1.0))
        dinv_ref[...] = dinv
        xb = x_ref[...] * (BN_C * bnw_ref[...]) + bnb_ref[...]
        z = jnp.dot(xb, w_ref[...], preferred_element_type=jnp.float32,
                    precision=lax.Precision.HIGHEST)
        z_ref[...] = z * dinv

    return pl.pallas_call(
        body,
        grid=(pl.cdiv(n, RB),),
        in_specs=[_row_spec(din), _full_spec((1, din)), _full_spec((1, din)),
                  pl.BlockSpec((NW, RB), lambda i: (0, i)),
                  _full_spec((din, h))],
        out_specs=[pl.BlockSpec((RB, 1), lambda i: (i, 0)), _row_spec(h)],
        out_shape=[jax.ShapeDtypeStruct((n, 1), jnp.float32),
                   jax.ShapeDtypeStruct((n, h), jnp.float32)],
    )


@functools.lru_cache(maxsize=None)
def _make_tc_mid(n, h):
    def body(sp_ref, z_ref, dinv_ref, b_ref, w_ref, h_ref, znext_ref):
        dinv = dinv_ref[...]
        g = (sp_ref[0] + sp_ref[1] + z_ref[...]) * dinv + b_ref[...]
        hv = jnp.maximum(g, 0.0)
        h_ref[...] = hv
        znext = jnp.dot(hv, w_ref[...], preferred_element_type=jnp.float32,
                        precision=lax.Precision.HIGHEST)
        znext_ref[...] = znext * dinv

    return pl.pallas_call(
        body,
        grid=(pl.cdiv(n, RB),),
        in_specs=[_part_spec(h), _row_spec(h), pl.BlockSpec((RB, 1), lambda i: (i, 0)),
                  _full_spec((1, h)), _full_spec((h, h))],
        out_specs=[_row_spec(h), _row_spec(h)],
        out_shape=[jax.ShapeDtypeStruct((n, h), jnp.float32),
                   jax.ShapeDtypeStruct((n, h), jnp.float32)],
    )


@functools.lru_cache(maxsize=None)
def _make_tc4(n, h, dout):
    def body(sp_ref, z_ref, dinv_ref, b_ref, bn2w_ref, bn2b_ref, wout_ref,
             h1_ref, h2_ref, h3_ref, z4_ref):
        dinv = dinv_ref[...]
        g = (sp_ref[0] + sp_ref[1] + z_ref[...]) * dinv + b_ref[...]
        h3 = jnp.maximum(g, 0.0)
        h3_ref[...] = h3
        z4 = jnp.zeros((RB, dout), jnp.float32)
        for k, hk in enumerate((h1_ref[...], h2_ref[...], h3)):
            xk = hk * (BN_C * bn2w_ref[:, k * h:(k + 1) * h]) + bn2b_ref[:, k * h:(k + 1) * h]
            z4 = z4 + jnp.dot(xk, wout_ref[k * h:(k + 1) * h, :],
                              preferred_element_type=jnp.float32,
                              precision=lax.Precision.HIGHEST)
        z4_ref[...] = jnp.concatenate(
            [z4 * dinv, jnp.zeros((RB, h - dout), jnp.float32)], axis=1)

    return pl.pallas_call(
        body,
        grid=(pl.cdiv(n, RB),),
        in_specs=[_part_spec(h), _row_spec(h), pl.BlockSpec((RB, 1), lambda i: (i, 0)),
                  _full_spec((1, h)), _full_spec((1, 3 * h)), _full_spec((1, 3 * h)),
                  _full_spec((3 * h, dout)), _row_spec(h), _row_spec(h)],
        out_specs=[_row_spec(h), _row_spec(h)],
        out_shape=[jax.ShapeDtypeStruct((n, h), jnp.float32),
                   jax.ShapeDtypeStruct((n, h), jnp.float32)],
    )


@functools.lru_cache(maxsize=None)
def _make_tc5(n, h, dout):
    def body(sp_ref, z_ref, dinv_ref, b_ref, out_ref):
        s = (sp_ref[0] + sp_ref[1] + z_ref[...]) * dinv_ref[...]
        out_ref[...] = s[:, :dout] + b_ref[...]

    return pl.pallas_call(
        body,
        grid=(pl.cdiv(n, RB),),
        in_specs=[_part_spec(h), _row_spec(h),
                  pl.BlockSpec((RB, 1), lambda i: (i, 0)), _full_spec((1, dout))],
        out_specs=_row_spec(dout),
        out_shape=jax.ShapeDtypeStruct((n, dout), jnp.float32),
    )


def kernel(x, edge_index, percent, ricci_curvature, W1, b1, W2, b2, W3, b3,
           Wout, bout, bn1_w, bn1_b, bn2_w, bn2_b):
    n, din = x.shape
    e = edge_index.shape[1]
    h = W1.shape[1]
    dout = Wout.shape[1]

    src = edge_index[0]
    dst = edge_index[1]
    zeros_h = jnp.zeros((CH, h), jnp.float32)

    b1r = b1.reshape(1, h)
    b2r = b2.reshape(1, h)
    b3r = b3.reshape(1, h)
    boutr = bout.reshape(1, dout)
    bn1wr = bn1_w.reshape(1, din)
    bn1br = bn1_b.reshape(1, din)
    bn2wr = bn2_w.reshape(1, 3 * h)
    bn2br = bn2_b.reshape(1, 3 * h)

    degp = _make_deg(n, e)(dst).reshape(NW, _npad(n))
    dinv, z1 = _make_tc1(n, din, h)(x, bn1wr, bn1br, degp, W1)
    prop_h = _make_prop(n, e, h)
    s1 = prop_h(z1, src, dst, zeros_h)
    h1, z2 = _make_tc_mid(n, h)(s1, z1, dinv, b1r, W2)
    s2 = prop_h(z2, src, dst, zeros_h)
    h2, z3 = _make_tc_mid(n, h)(s2, z2, dinv, b2r, W3)
    s3 = prop_h(z3, src, dst, zeros_h)
    h3, z4 = _make_tc4(n, h, dout)(s3, z3, dinv, b3r, bn2wr, bn2br, Wout, h1, h2)
    s4 = prop_h(z4, src, dst, zeros_h)
    out = _make_tc5(n, h, dout)(s4, z4, dinv, boutr)
    return out, h1, h2, h3


# speedup vs baseline: 22.8010x; 2.2789x over previous
import functools

import jax
import jax.numpy as jnp
from jax import lax
from jax.experimental import pallas as pl
from jax.experimental.pallas import tpu as pltpu
from jax.experimental.pallas import tpu_sc as plsc

NC = 2
NS = 16
NW = NC * NS
CH = 80
DEGW = 8
RB = 1280
BN_C = 1.0 / (1.0 + 1e-5) ** 0.5


def _sc_mesh():
    return plsc.VectorSubcoreMesh(core_axis_name="c", subcore_axis_name="s")


def _npad(n):
    return ((n + NS * CH - 1) // (NS * CH)) * (NS * CH)


@functools.lru_cache(maxsize=None)
def _make_deg(n, e):
    ept = e // NW
    npad = _npad(n)
    grows = npad // 128

    @functools.partial(
        pl.kernel,
        out_type=jax.ShapeDtypeStruct((NW, grows, 128), jnp.float32),
        mesh=_sc_mesh(),
        compiler_params=pltpu.CompilerParams(needs_layout_passes=False),
        scratch_types=[
            pltpu.VMEM((ept,), jnp.int32),
            pltpu.VMEM((grows, 128), jnp.float32),
        ],
    )
    def deg_kernel(dst_hbm, out_hbm, idx_v, deg_v):
        cid = lax.axis_index("c")
        sid = lax.axis_index("s")
        wid = sid * NC + cid

        def zero(i, carry):
            for t in range(8):
                deg_v[i, pl.ds(t * 16, 16)] = jnp.zeros((16,), jnp.float32)
            return carry

        lax.fori_loop(0, grows, zero, 0)
        pltpu.sync_copy(dst_hbm.at[pl.ds(pl.multiple_of(wid * ept, 8), ept)], idx_v)
        ones16 = jnp.ones((16,), jnp.float32)

        def body(j, carry):
            for t in range(5):
                idx = idx_v[pl.ds(j * 80 + t * 16, 16)]
                plsc.addupdate_scatter(deg_v, [idx >> 7, idx & 127], ones16)
            return carry

        lax.fori_loop(0, ept // 80, body, 0)
        pltpu.sync_copy(deg_v, out_hbm.at[wid])

    return deg_kernel


@functools.lru_cache(maxsize=None)
def _make_prop(n, e, d):
    ept = e // NW
    nchunk = ept // CH
    assert nchunk % 2 == 1 and nchunk >= 3
    npad = _npad(n)
    slab = npad // NS

    @functools.partial(
        pl.kernel,
        out_type=jax.ShapeDtypeStruct((NC, npad, d), jnp.float32),
        mesh=_sc_mesh(),
        scratch_types=[
            pltpu.VMEM((ept,), jnp.int32),
            pltpu.VMEM((nchunk, CH), jnp.int32),
            pltpu.VMEM((CH, d), jnp.float32),
            pltpu.VMEM((CH, d), jnp.float32),
            pltpu.VMEM_SHARED((npad, d), jnp.float32),
            pltpu.SemaphoreType.DMA,
            pltpu.SemaphoreType.DMA,
            pltpu.SemaphoreType.DMA,
        ],
    )
    def prop_kernel(z_hbm, src_hbm, dst_hbm, zeros_hbm, out_hbm,
                    sidx_all, didx_all, row_a, row_b, acc, sem_i, sem_a, sem_b):
        cid = lax.axis_index("c")
        sid = lax.axis_index("s")
        wid = sid * NC + cid
        nwb = slab // CH

        ld_s = pltpu.async_copy(
            src_hbm.at[pl.ds(pl.multiple_of(wid * ept, 8), ept)], sidx_all, sem_i)
        ld_d = pltpu.async_copy(dst_hbm.at[wid], didx_all, sem_i)
        pltpu.sync_copy(zeros_hbm, row_b)

        def zinit(k, carry):
            off = pl.multiple_of(sid * slab + k * CH, 8)
            pltpu.sync_copy(row_b, acc.at[pl.ds(off, CH)])
            return carry

        lax.fori_loop(0, nwb, zinit, 0)
        ld_s.wait()
        ld_d.wait()
        plsc.subcore_barrier()

        def wait_a():
            pltpu.make_async_copy(z_hbm.at[pl.ds(0, CH)], row_a, sem_a).wait()

        def wait_b():
            pltpu.make_async_copy(z_hbm.at[pl.ds(0, CH)], row_b, sem_b).wait()

        pltpu.async_copy(z_hbm.at[sidx_all.at[pl.ds(0, CH)]], row_a, sem_a)

        def body(i, carry):
            j = 2 * i
            pltpu.async_copy(z_hbm.at[sidx_all.at[pl.ds((j + 1) * CH, CH)]], row_b, sem_b)
            wait_a()
            pltpu.sync_copy(row_a, acc.at[didx_all.at[j]], add=True)
            pltpu.async_copy(z_hbm.at[sidx_all.at[pl.ds((j + 2) * CH, CH)]], row_a, sem_a)
            wait_b()
            pltpu.sync_copy(row_b, acc.at[didx_all.at[j + 1]], add=True)
            return carry

        lax.fori_loop(0, nchunk // 2, body, 0)
        wait_a()
        pltpu.sync_copy(row_a, acc.at[didx_all.at[nchunk - 1]], add=True)
        plsc.subcore_barrier()

        def wback(k, carry):
            off = pl.multiple_of(sid * slab + k * CH, 8)
            pltpu.sync_copy(acc.at[pl.ds(off, CH)], row_a)
            pltpu.sync_copy(row_a, out_hbm.at[cid, pl.ds(off, CH)])
            return carry

        lax.fori_loop(0, nwb, wback, 0)

    return prop_kernel


def _row_spec(d):
    return pl.BlockSpec((RB, d), lambda i: (i, 0))


def _full_spec(shape):
    nd = len(shape)
    return pl.BlockSpec(shape, lambda i: (0,) * nd)


def _part_spec(d):
    return pl.BlockSpec((NC, RB, d), lambda i: (0, i, 0))


@functools.lru_cache(maxsize=None)
def _make_tc1(n, din, h):
    G = RB // 128

    def body(x_ref, bnw_ref, bnb_ref, degp_ref, w_ref, dinv_ref, z_ref):
        dsum = jnp.sum(degp_ref[...], axis=0, keepdims=True) + 1.0
        io = lax.broadcasted_iota(jnp.int32, (128, 128), 0)
        ic = lax.broadcasted_iota(jnp.int32, (128, 128), 1)
        eye = (io == ic).astype(jnp.float32)
        onesc = jnp.ones((128, 1), jnp.float32)
        cols = [jnp.dot(eye * dsum[:, g * 128:(g + 1) * 128], onesc,
                        preferred_element_type=jnp.float32,
                        precision=lax.Precision.HIGHEST) for g in range(G)]
        deg = jnp.concatenate(cols, axis=0)
        dinv = lax.rsqrt(jnp.maximum(deg, 1.0))
        dinv_ref[...] = dinv
        xb = x_ref[...] * (BN_C * bnw_ref[...]) + bnb_ref[...]
        z = jnp.dot(xb, w_ref[...], preferred_element_type=jnp.float32,
                    precision=lax.Precision.HIGHEST)
        z_ref[...] = z * dinv

    return pl.pallas_call(
        body,
        grid=(pl.cdiv(n, RB),),
        in_specs=[_row_spec(din), _full_spec((1, din)), _full_spec((1, din)),
                  pl.BlockSpec((NW, RB), lambda i: (0, i)),
                  _full_spec((din, h))],
        out_specs=[pl.BlockSpec((RB, 1), lambda i: (i, 0)), _row_spec(h)],
        out_shape=[jax.ShapeDtypeStruct((n, 1), jnp.float32),
                   jax.ShapeDtypeStruct((n, h), jnp.float32)],
    )


@functools.lru_cache(maxsize=None)
def _make_tc_mid(n, h):
    def body(sp_ref, z_ref, dinv_ref, b_ref, w_ref, h_ref, znext_ref):
        dinv = dinv_ref[...]
        g = (sp_ref[0] + sp_ref[1] + z_ref[...]) * dinv + b_ref[...]
        hv = jnp.maximum(g, 0.0)
        h_ref[...] = hv
        znext = jnp.dot(hv, w_ref[...], preferred_element_type=jnp.float32,
                        precision=lax.Precision.HIGHEST)
        znext_ref[...] = znext * dinv

    return pl.pallas_call(
        body,
        grid=(pl.cdiv(n, RB),),
        in_specs=[_part_spec(h), _row_spec(h), pl.BlockSpec((RB, 1), lambda i: (i, 0)),
                  _full_spec((1, h)), _full_spec((h, h))],
        out_specs=[_row_spec(h), _row_spec(h)],
        out_shape=[jax.ShapeDtypeStruct((n, h), jnp.float32),
                   jax.ShapeDtypeStruct((n, h), jnp.float32)],
    )


@functools.lru_cache(maxsize=None)
def _make_tc4(n, h, dout):
    def body(sp_ref, z_ref, dinv_ref, b_ref, bn2w_ref, bn2b_ref, wout_ref,
             h1_ref, h2_ref, h3_ref, z4_ref):
        dinv = dinv_ref[...]
        g = (sp_ref[0] + sp_ref[1] + z_ref[...]) * dinv + b_ref[...]
        h3 = jnp.maximum(g, 0.0)
        h3_ref[...] = h3
        z4 = jnp.zeros((RB, dout), jnp.float32)
        for k, hk in enumerate((h1_ref[...], h2_ref[...], h3)):
            xk = hk * (BN_C * bn2w_ref[:, k * h:(k + 1) * h]) + bn2b_ref[:, k * h:(k + 1) * h]
            z4 = z4 + jnp.dot(xk, wout_ref[k * h:(k + 1) * h, :],
                              preferred_element_type=jnp.float32,
                              precision=lax.Precision.HIGHEST)
        z4_ref[...] = jnp.concatenate(
            [z4 * dinv, jnp.zeros((RB, h - dout), jnp.float32)], axis=1)

    return pl.pallas_call(
        body,
        grid=(pl.cdiv(n, RB),),
        in_specs=[_part_spec(h), _row_spec(h), pl.BlockSpec((RB, 1), lambda i: (i, 0)),
                  _full_spec((1, h)), _full_spec((1, 3 * h)), _full_spec((1, 3 * h)),
                  _full_spec((3 * h, dout)), _row_spec(h), _row_spec(h)],
        out_specs=[_row_spec(h), _row_spec(h)],
        out_shape=[jax.ShapeDtypeStruct((n, h), jnp.float32),
                   jax.ShapeDtypeStruct((n, h), jnp.float32)],
    )


@functools.lru_cache(maxsize=None)
def _make_tc5(n, h, dout):
    def body(sp_ref, z_ref, dinv_ref, b_ref, out_ref):
        s = (sp_ref[0] + sp_ref[1] + z_ref[...]) * dinv_ref[...]
        out_ref[...] = s[:, :dout] + b_ref[...]

    return pl.pallas_call(
        body,
        grid=(pl.cdiv(n, RB),),
        in_specs=[_part_spec(h), _row_spec(h),
                  pl.BlockSpec((RB, 1), lambda i: (i, 0)), _full_spec((1, dout))],
        out_specs=_row_spec(dout),
        out_shape=jax.ShapeDtypeStruct((n, dout), jnp.float32),
    )


def kernel(x, edge_index, percent, ricci_curvature, W1, b1, W2, b2, W3, b3,
           Wout, bout, bn1_w, bn1_b, bn2_w, bn2_b):
    n, din = x.shape
    e = edge_index.shape[1]
    h = W1.shape[1]
    dout = Wout.shape[1]

    ept = e // NW
    nchunk = ept // CH
    src = edge_index[0]
    dst = edge_index[1].reshape(NW, nchunk, CH)
    zeros_h = jnp.zeros((CH, h), jnp.float32)

    b1r = b1.reshape(1, h)
    b2r = b2.reshape(1, h)
    b3r = b3.reshape(1, h)
    boutr = bout.reshape(1, dout)
    bn1wr = bn1_w.reshape(1, din)
    bn1br = bn1_b.reshape(1, din)
    bn2wr = bn2_w.reshape(1, 3 * h)
    bn2br = bn2_b.reshape(1, 3 * h)

    degp = _make_deg(n, e)(edge_index[1]).reshape(NW, _npad(n))
    dinv, z1 = _make_tc1(n, din, h)(x, bn1wr, bn1br, degp, W1)
    prop_h = _make_prop(n, e, h)
    s1 = prop_h(z1, src, dst, zeros_h)
    h1, z2 = _make_tc_mid(n, h)(s1, z1, dinv, b1r, W2)
    s2 = prop_h(z2, src, dst, zeros_h)
    h2, z3 = _make_tc_mid(n, h)(s2, z2, dinv, b2r, W3)
    s3 = prop_h(z3, src, dst, zeros_h)
    h3, z4 = _make_tc4(n, h, dout)(s3, z3, dinv, b3r, bn2wr, bn2br, Wout, h1, h2)
    s4 = prop_h(z4, src, dst, zeros_h)
    out = _make_tc5(n, h, dout)(s4, z4, dinv, boutr)
    return out, h1, h2, h3


# speedup vs baseline: 24.7161x; 1.0840x over previous
import functools

import jax
import jax.numpy as jnp
from jax import lax
from jax.experimental import pallas as pl
from jax.experimental.pallas import tpu as pltpu
from jax.experimental.pallas import tpu_sc as plsc

NC = 2
NS = 16
NW = NC * NS
CH = 128
DEGW = 8
RB = 1280
BN_C = 1.0 / (1.0 + 1e-5) ** 0.5


def _sc_mesh():
    return plsc.VectorSubcoreMesh(core_axis_name="c", subcore_axis_name="s")


def _npad(n):
    return ((n + NS * CH - 1) // (NS * CH)) * (NS * CH)


@functools.lru_cache(maxsize=None)
def _make_deg(n, e):
    ept = e // NW
    npad = _npad(n)
    grows = npad // 128

    @functools.partial(
        pl.kernel,
        out_type=jax.ShapeDtypeStruct((NW, grows, 128), jnp.float32),
        mesh=_sc_mesh(),
        compiler_params=pltpu.CompilerParams(needs_layout_passes=False),
        scratch_types=[
            pltpu.VMEM((ept,), jnp.int32),
            pltpu.VMEM((grows, 128), jnp.float32),
        ],
    )
    def deg_kernel(dst_hbm, out_hbm, idx_v, deg_v):
        cid = lax.axis_index("c")
        sid = lax.axis_index("s")
        wid = sid * NC + cid

        def zero(i, carry):
            for t in range(8):
                deg_v[i, pl.ds(t * 16, 16)] = jnp.zeros((16,), jnp.float32)
            return carry

        lax.fori_loop(0, grows, zero, 0)
        pltpu.sync_copy(dst_hbm.at[pl.ds(pl.multiple_of(wid * ept, 8), ept)], idx_v)
        ones16 = jnp.ones((16,), jnp.float32)

        def body(j, carry):
            for t in range(5):
                idx = idx_v[pl.ds(j * 80 + t * 16, 16)]
                plsc.addupdate_scatter(deg_v, [idx >> 7, idx & 127], ones16)
            return carry

        lax.fori_loop(0, ept // 80, body, 0)
        pltpu.sync_copy(deg_v, out_hbm.at[wid])

    return deg_kernel


@functools.lru_cache(maxsize=None)
def _make_prop(n, e, d):
    ncht = e // CH
    tbase = ncht // NW
    rem = ncht % NW
    tmax = tbase + (1 if rem else 0)
    assert tbase % 2 == 0 and tbase >= 4
    npad = _npad(n)
    slab = npad // NS
    assert slab % CH == 0

    @functools.partial(
        pl.kernel,
        out_type=jax.ShapeDtypeStruct((NC, npad, d), jnp.float32),
        mesh=_sc_mesh(),
        scratch_types=[
            pltpu.VMEM((tmax * CH,), jnp.int32),
            pltpu.VMEM((CH,), jnp.int32),
            pltpu.VMEM((CH,), jnp.int32),
            pltpu.VMEM((CH, d), jnp.float32),
            pltpu.VMEM((CH, d), jnp.float32),
            pltpu.VMEM_SHARED((npad, d), jnp.float32),
            pltpu.SemaphoreType.DMA,
            pltpu.SemaphoreType.DMA,
        ],
    )
    def prop_kernel(z_hbm, src_hbm, dst_hbm, zeros_hbm, out_hbm,
                    sidx_all, didx_a, didx_b, row_a, row_b, acc, sem_a, sem_b):
        cid = lax.axis_index("c")
        sid = lax.axis_index("s")
        wid = sid * NC + cid
        cb = tbase * wid + jnp.minimum(wid, rem)
        tw = tbase + (wid < rem).astype(jnp.int32)
        nwb = slab // CH

        ld_s = pltpu.async_copy(
            src_hbm.at[pl.ds(pl.multiple_of(cb * CH, 8), tmax * CH)],
            sidx_all, sem_a)
        pltpu.sync_copy(zeros_hbm, row_b)

        def zinit(k, carry):
            off = pl.multiple_of(sid * slab + k * CH, 8)
            pltpu.sync_copy(row_b, acc.at[pl.ds(off, CH)])
            return carry

        lax.fori_loop(0, nwb, zinit, 0)
        ld_s.wait()
        plsc.subcore_barrier()

        def gather_a(j):
            pltpu.async_copy(dst_hbm.at[cb + j], didx_a, sem_a)
            pltpu.async_copy(z_hbm.at[sidx_all.at[pl.ds(j * CH, CH)]],
                             row_a, sem_a)

        def gather_b(j):
            pltpu.async_copy(dst_hbm.at[cb + j], didx_b, sem_b)
            pltpu.async_copy(z_hbm.at[sidx_all.at[pl.ds(j * CH, CH)]],
                             row_b, sem_b)

        def wait_ab(row, didx, sem):
            pltpu.make_async_copy(z_hbm.at[pl.ds(0, CH)], row, sem).wait()
            pltpu.make_async_copy(dst_hbm.at[0], didx, sem).wait()

        gather_a(0)

        def body(i, carry):
            j = 2 * i
            gather_b(j + 1)
            wait_ab(row_a, didx_a, sem_a)
            pltpu.sync_copy(row_a, acc.at[didx_a], add=True)

            @pl.when(j + 2 < tw)
            def _():
                gather_a(j + 2)

            wait_ab(row_b, didx_b, sem_b)
            pltpu.sync_copy(row_b, acc.at[didx_b], add=True)
            return carry

        lax.fori_loop(0, tbase // 2, body, 0)

        @pl.when(tw > tbase)
        def _():
            wait_ab(row_a, didx_a, sem_a)
            pltpu.sync_copy(row_a, acc.at[didx_a], add=True)

        plsc.subcore_barrier()

        def wback(k, carry):
            off = pl.multiple_of(sid * slab + k * CH, 8)
            pltpu.sync_copy(acc.at[pl.ds(off, CH)], row_a)
            pltpu.sync_copy(row_a, out_hbm.at[cid, pl.ds(off, CH)])
            return carry

        lax.fori_loop(0, nwb, wback, 0)

    return prop_kernel


def _row_spec(d):
    return pl.BlockSpec((RB, d), lambda i: (i, 0))


def _full_spec(shape):
    nd = len(shape)
    return pl.BlockSpec(shape, lambda i: (0,) * nd)


def _part_spec(d):
    return pl.BlockSpec((NC, RB, d), lambda i: (0, i, 0))


@functools.lru_cache(maxsize=None)
def _make_tc1(n, din, h):
    G = RB // 128

    def body(x_ref, bnw_ref, bnb_ref, degp_ref, w_ref, dinv_ref, z_ref):
        dsum = jnp.sum(degp_ref[...], axis=0, keepdims=True) + 1.0
        io = lax.broadcasted_iota(jnp.int32, (128, 128), 0)
        ic = lax.broadcasted_iota(jnp.int32, (128, 128), 1)
        eye = (io == ic).astype(jnp.float32)
        onesc = jnp.ones((128, 1), jnp.float32)
        cols = [jnp.dot(eye * dsum[:, g * 128:(g + 1) * 128], onesc,
                        preferred_element_type=jnp.float32,
                        precision=lax.Precision.HIGHEST) for g in range(G)]
        deg = jnp.concatenate(cols, axis=0)
        dinv = lax.rsqrt(jnp.maximum(deg, 1.0))
        dinv_ref[...] = dinv
        xb = x_ref[...] * (BN_C * bnw_ref[...]) + bnb_ref[...]
        z = jnp.dot(xb, w_ref[...], preferred_element_type=jnp.float32,
                    precision=lax.Precision.HIGHEST)
        z_ref[...] = z * dinv

    return pl.pallas_call(
        body,
        grid=(pl.cdiv(n, RB),),
        in_specs=[_row_spec(din), _full_spec((1, din)), _full_spec((1, din)),
                  pl.BlockSpec((NW, RB), lambda i: (0, i)),
                  _full_spec((din, h))],
        out_specs=[pl.BlockSpec((RB, 1), lambda i: (i, 0)), _row_spec(h)],
        out_shape=[jax.ShapeDtypeStruct((n, 1), jnp.float32),
                   jax.ShapeDtypeStruct((n, h), jnp.float32)],
    )


@functools.lru_cache(maxsize=None)
def _make_tc_mid(n, h):
    def body(sp_ref, z_ref, dinv_ref, b_ref, w_ref, h_ref, znext_ref):
        dinv = dinv_ref[...]
        g = (sp_ref[0] + sp_ref[1] + z_ref[...]) * dinv + b_ref[...]
        hv = jnp.maximum(g, 0.0)
        h_ref[...] = hv
        znext = jnp.dot(hv, w_ref[...], preferred_element_type=jnp.float32,
                        precision=lax.Precision.HIGHEST)
        znext_ref[...] = znext * dinv

    return pl.pallas_call(
        body,
        grid=(pl.cdiv(n, RB),),
        in_specs=[_part_spec(h), _row_spec(h), pl.BlockSpec((RB, 1), lambda i: (i, 0)),
                  _full_spec((1, h)), _full_spec((h, h))],
        out_specs=[_row_spec(h), _row_spec(h)],
        out_shape=[jax.ShapeDtypeStruct((n, h), jnp.float32),
                   jax.ShapeDtypeStruct((n, h), jnp.float32)],
    )


@functools.lru_cache(maxsize=None)
def _make_tc4(n, h, dout):
    def body(sp_ref, z_ref, dinv_ref, b_ref, bn2w_ref, bn2b_ref, wout_ref,
             h1_ref, h2_ref, h3_ref, z4_ref):
        dinv = dinv_ref[...]
        g = (sp_ref[0] + sp_ref[1] + z_ref[...]) * dinv + b_ref[...]
        h3 = jnp.maximum(g, 0.0)
        h3_ref[...] = h3
        z4 = jnp.zeros((RB, dout), jnp.float32)
        for k, hk in enumerate((h1_ref[...], h2_ref[...], h3)):
            xk = hk * (BN_C * bn2w_ref[:, k * h:(k + 1) * h]) + bn2b_ref[:, k * h:(k + 1) * h]
            z4 = z4 + jnp.dot(xk, wout_ref[k * h:(k + 1) * h, :],
                              preferred_element_type=jnp.float32,
                              precision=lax.Precision.HIGHEST)
        z4_ref[...] = jnp.concatenate(
            [z4 * dinv, jnp.zeros((RB, h - dout), jnp.float32)], axis=1)

    return pl.pallas_call(
        body,
        grid=(pl.cdiv(n, RB),),
        in_specs=[_part_spec(h), _row_spec(h), pl.BlockSpec((RB, 1), lambda i: (i, 0)),
                  _full_spec((1, h)), _full_spec((1, 3 * h)), _full_spec((1, 3 * h)),
                  _full_spec((3 * h, dout)), _row_spec(h), _row_spec(h)],
        out_specs=[_row_spec(h), _row_spec(h)],
        out_shape=[jax.ShapeDtypeStruct((n, h), jnp.float32),
                   jax.ShapeDtypeStruct((n, h), jnp.float32)],
    )


@functools.lru_cache(maxsize=None)
def _make_tc5(n, h, dout):
    def body(sp_ref, z_ref, dinv_ref, b_ref, out_ref):
        s = (sp_ref[0] + sp_ref[1] + z_ref[...]) * dinv_ref[...]
        out_ref[...] = s[:, :dout] + b_ref[...]

    return pl.pallas_call(
        body,
        grid=(pl.cdiv(n, RB),),
        in_specs=[_part_spec(h), _row_spec(h),
                  pl.BlockSpec((RB, 1), lambda i: (i, 0)), _full_spec((1, dout))],
        out_specs=_row_spec(dout),
        out_shape=jax.ShapeDtypeStruct((n, dout), jnp.float32),
    )


def kernel(x, edge_index, percent, ricci_curvature, W1, b1, W2, b2, W3, b3,
           Wout, bout, bn1_w, bn1_b, bn2_w, bn2_b):
    n, din = x.shape
    e = edge_index.shape[1]
    h = W1.shape[1]
    dout = Wout.shape[1]

    ncht = e // CH
    tmax = ncht // NW + (1 if ncht % NW else 0)
    epad = (ncht + NW) * CH
    src = jnp.pad(edge_index[0], (0, epad - e))
    dst = jnp.pad(edge_index[1], (0, epad - e)).reshape(epad // CH, CH)
    zeros_h = jnp.zeros((CH, h), jnp.float32)

    b1r = b1.reshape(1, h)
    b2r = b2.reshape(1, h)
    b3r = b3.reshape(1, h)
    boutr = bout.reshape(1, dout)
    bn1wr = bn1_w.reshape(1, din)
    bn1br = bn1_b.reshape(1, din)
    bn2wr = bn2_w.reshape(1, 3 * h)
    bn2br = bn2_b.reshape(1, 3 * h)

    degp = _make_deg(n, e)(edge_index[1]).reshape(NW, _npad(n))
    dinv, z1 = _make_tc1(n, din, h)(x, bn1wr, bn1br, degp, W1)
    prop_h = _make_prop(n, e, h)
    s1 = prop_h(z1, src, dst, zeros_h)
    h1, z2 = _make_tc_mid(n, h)(s1, z1, dinv, b1r, W2)
    s2 = prop_h(z2, src, dst, zeros_h)
    h2, z3 = _make_tc_mid(n, h)(s2, z2, dinv, b2r, W3)
    s3 = prop_h(z3, src, dst, zeros_h)
    h3, z4 = _make_tc4(n, h, dout)(s3, z3, dinv, b3r, bn2wr, bn2br, Wout, h1, h2)
    s4 = prop_h(z4, src, dst, zeros_h)
    out = _make_tc5(n, h, dout)(s4, z4, dinv, boutr)
    return out, h1, h2, h3


# speedup vs baseline: 25.2287x; 1.0207x over previous
import functools

import jax
import jax.numpy as jnp
from jax import lax
from jax.experimental import pallas as pl
from jax.experimental.pallas import tpu as pltpu
from jax.experimental.pallas import tpu_sc as plsc

NC = 2
NS = 16
NW = NC * NS
CH = 128
DEGW = 8
RB = 1280
BN_C = 1.0 / (1.0 + 1e-5) ** 0.5


def _sc_mesh():
    return plsc.VectorSubcoreMesh(core_axis_name="c", subcore_axis_name="s")


def _npad(n):
    return ((n + NS * CH - 1) // (NS * CH)) * (NS * CH)


@functools.lru_cache(maxsize=None)
def _make_deg(n, e):
    ept = e // NW
    npad = _npad(n)
    grows = npad // 128

    @functools.partial(
        pl.kernel,
        out_type=jax.ShapeDtypeStruct((NW, grows, 128), jnp.float32),
        mesh=_sc_mesh(),
        compiler_params=pltpu.CompilerParams(needs_layout_passes=False),
        scratch_types=[
            pltpu.VMEM((ept,), jnp.int32),
            pltpu.VMEM((grows, 128), jnp.float32),
        ],
    )
    def deg_kernel(dst_hbm, out_hbm, idx_v, deg_v):
        cid = lax.axis_index("c")
        sid = lax.axis_index("s")
        wid = sid * NC + cid

        def zero(i, carry):
            for t in range(8):
                deg_v[i, pl.ds(t * 16, 16)] = jnp.zeros((16,), jnp.float32)
            return carry

        lax.fori_loop(0, grows, zero, 0)
        pltpu.sync_copy(dst_hbm.at[pl.ds(pl.multiple_of(wid * ept, 8), ept)], idx_v)
        ones16 = jnp.ones((16,), jnp.float32)

        def body(j, carry):
            for t in range(5):
                idx = idx_v[pl.ds(j * 80 + t * 16, 16)]
                plsc.addupdate_scatter(deg_v, [idx >> 7, idx & 127], ones16)
            return carry

        lax.fori_loop(0, ept // 80, body, 0)
        pltpu.sync_copy(deg_v, out_hbm.at[wid])

    return deg_kernel


@functools.lru_cache(maxsize=None)
def _make_prop(n, e, d):
    ncht = e // CH
    tbase = ncht // NW
    rem = ncht % NW
    tmax = tbase + (1 if rem else 0)
    assert tbase % 2 == 0 and tbase >= 4
    npad = _npad(n)
    slab = npad // NS
    assert slab % CH == 0

    @functools.partial(
        pl.kernel,
        out_type=jax.ShapeDtypeStruct((NC, npad, d), jnp.float32),
        mesh=_sc_mesh(),
        scratch_types=[
            pltpu.VMEM((tmax * CH,), jnp.int32),
            pltpu.VMEM((CH,), jnp.int32),
            pltpu.VMEM((CH,), jnp.int32),
            pltpu.VMEM((CH, d), jnp.float32),
            pltpu.VMEM((CH, d), jnp.float32),
            pltpu.VMEM_SHARED((npad, d), jnp.float32),
            pltpu.SemaphoreType.DMA,
            pltpu.SemaphoreType.DMA,
        ],
    )
    def prop_kernel(z_hbm, src_hbm, dst_hbm, zeros_hbm, out_hbm,
                    sidx_all, didx_a, didx_b, row_a, row_b, acc, sem_a, sem_b):
        cid = lax.axis_index("c")
        sid = lax.axis_index("s")
        wid = sid * NC + cid
        cb = tbase * wid + jnp.minimum(wid, rem)
        tw = tbase + (wid < rem).astype(jnp.int32)
        nwb = slab // CH

        ld_s = pltpu.async_copy(
            src_hbm.at[pl.ds(pl.multiple_of(cb * CH, 8), tmax * CH)],
            sidx_all, sem_a)
        pltpu.sync_copy(zeros_hbm, row_b)

        def zinit(k, carry):
            off = pl.multiple_of(sid * slab + k * CH, 8)
            pltpu.sync_copy(row_b, acc.at[pl.ds(off, CH)])
            return carry

        lax.fori_loop(0, nwb, zinit, 0)
        ld_s.wait()
        plsc.subcore_barrier()

        def gather_a(j):
            pltpu.async_copy(dst_hbm.at[cb + j], didx_a, sem_a)
            pltpu.async_copy(z_hbm.at[sidx_all.at[pl.ds(j * CH, CH)]],
                             row_a, sem_a)

        def gather_b(j):
            pltpu.async_copy(dst_hbm.at[cb + j], didx_b, sem_b)
            pltpu.async_copy(z_hbm.at[sidx_all.at[pl.ds(j * CH, CH)]],
                             row_b, sem_b)

        def wait_ab(row, didx, sem):
            pltpu.make_async_copy(z_hbm.at[pl.ds(0, CH)], row, sem).wait()
            pltpu.make_async_copy(dst_hbm.at[0], didx, sem).wait()

        gather_a(0)

        def body(i, carry):
            j = 2 * i
            gather_b(j + 1)
            wait_ab(row_a, didx_a, sem_a)
            pltpu.sync_copy(row_a, acc.at[didx_a], add=True)

            @pl.when(j + 2 < tw)
            def _():
                gather_a(j + 2)

            wait_ab(row_b, didx_b, sem_b)
            pltpu.sync_copy(row_b, acc.at[didx_b], add=True)
            return carry

        lax.fori_loop(0, tbase // 2, body, 0)

        @pl.when(tw > tbase)
        def _():
            wait_ab(row_a, didx_a, sem_a)
            pltpu.sync_copy(row_a, acc.at[didx_a], add=True)

        plsc.subcore_barrier()

        def wback(k, carry):
            off = pl.multiple_of(sid * slab + k * CH, 8)
            pltpu.sync_copy(acc.at[pl.ds(off, CH)], row_a)
            pltpu.sync_copy(row_a, out_hbm.at[cid, pl.ds(off, CH)])
            return carry

        lax.fori_loop(0, nwb, wback, 0)

    return prop_kernel


def _row_spec(d):
    return pl.BlockSpec((RB, d), lambda i: (i, 0))


def _full_spec(shape):
    nd = len(shape)
    return pl.BlockSpec(shape, lambda i: (0,) * nd)


def _part_spec(d):
    return pl.BlockSpec((NC, RB, d), lambda i: (0, i, 0))


@functools.lru_cache(maxsize=None)
def _make_tc1(n, din, h):
    G = RB // 128

    def body(x_ref, bnw_ref, bnb_ref, degp_ref, w_ref, dinv_ref, z_ref):
        dsum = jnp.sum(degp_ref[...], axis=0, keepdims=True) + 1.0
        io = lax.broadcasted_iota(jnp.int32, (128, 128), 0)
        ic = lax.broadcasted_iota(jnp.int32, (128, 128), 1)
        eye = (io == ic).astype(jnp.float32)
        onesc = jnp.ones((128, 1), jnp.float32)
        cols = [jnp.dot(eye * dsum[:, g * 128:(g + 1) * 128], onesc,
                        preferred_element_type=jnp.float32) for g in range(G)]
        deg = jnp.concatenate(cols, axis=0)
        dinv = lax.rsqrt(jnp.maximum(deg, 1.0))
        dinv_ref[...] = dinv
        xb = x_ref[...] * (BN_C * bnw_ref[...]) + bnb_ref[...]
        z = jnp.dot(xb, w_ref[...], preferred_element_type=jnp.float32)
        z_ref[...] = z * dinv

    return pl.pallas_call(
        body,
        grid=(pl.cdiv(n, RB),),
        in_specs=[_row_spec(din), _full_spec((1, din)), _full_spec((1, din)),
                  pl.BlockSpec((NW, RB), lambda i: (0, i)),
                  _full_spec((din, h))],
        out_specs=[pl.BlockSpec((RB, 1), lambda i: (i, 0)), _row_spec(h)],
        out_shape=[jax.ShapeDtypeStruct((n, 1), jnp.float32),
                   jax.ShapeDtypeStruct((n, h), jnp.float32)],
    )


@functools.lru_cache(maxsize=None)
def _make_tc_mid(n, h):
    def body(sp_ref, z_ref, dinv_ref, b_ref, w_ref, h_ref, znext_ref):
        dinv = dinv_ref[...]
        g = (sp_ref[0] + sp_ref[1] + z_ref[...]) * dinv + b_ref[...]
        hv = jnp.maximum(g, 0.0)
        h_ref[...] = hv
        znext = jnp.dot(hv, w_ref[...], preferred_element_type=jnp.float32)
        znext_ref[...] = znext * dinv

    return pl.pallas_call(
        body,
        grid=(pl.cdiv(n, RB),),
        in_specs=[_part_spec(h), _row_spec(h), pl.BlockSpec((RB, 1), lambda i: (i, 0)),
                  _full_spec((1, h)), _full_spec((h, h))],
        out_specs=[_row_spec(h), _row_spec(h)],
        out_shape=[jax.ShapeDtypeStruct((n, h), jnp.float32),
                   jax.ShapeDtypeStruct((n, h), jnp.float32)],
    )


@functools.lru_cache(maxsize=None)
def _make_tc4(n, h, dout):
    def body(sp_ref, z_ref, dinv_ref, b_ref, bn2w_ref, bn2b_ref, wout_ref,
             h1_ref, h2_ref, h3_ref, z4_ref):
        dinv = dinv_ref[...]
        g = (sp_ref[0] + sp_ref[1] + z_ref[...]) * dinv + b_ref[...]
        h3 = jnp.maximum(g, 0.0)
        h3_ref[...] = h3
        z4 = jnp.zeros((RB, dout), jnp.float32)
        for k, hk in enumerate((h1_ref[...], h2_ref[...], h3)):
            xk = hk * (BN_C * bn2w_ref[:, k * h:(k + 1) * h]) + bn2b_ref[:, k * h:(k + 1) * h]
            z4 = z4 + jnp.dot(xk, wout_ref[k * h:(k + 1) * h, :],
                              preferred_element_type=jnp.float32)
        z4_ref[...] = jnp.concatenate(
            [z4 * dinv, jnp.zeros((RB, h - dout), jnp.float32)], axis=1)

    return pl.pallas_call(
        body,
        grid=(pl.cdiv(n, RB),),
        in_specs=[_part_spec(h), _row_spec(h), pl.BlockSpec((RB, 1), lambda i: (i, 0)),
                  _full_spec((1, h)), _full_spec((1, 3 * h)), _full_spec((1, 3 * h)),
                  _full_spec((3 * h, dout)), _row_spec(h), _row_spec(h)],
        out_specs=[_row_spec(h), _row_spec(h)],
        out_shape=[jax.ShapeDtypeStruct((n, h), jnp.float32),
                   jax.ShapeDtypeStruct((n, h), jnp.float32)],
    )


@functools.lru_cache(maxsize=None)
def _make_tc5(n, h, dout):
    def body(sp_ref, z_ref, dinv_ref, b_ref, out_ref):
        s = (sp_ref[0] + sp_ref[1] + z_ref[...]) * dinv_ref[...]
        out_ref[...] = s[:, :dout] + b_ref[...]

    return pl.pallas_call(
        body,
        grid=(pl.cdiv(n, RB),),
        in_specs=[_part_spec(h), _row_spec(h),
                  pl.BlockSpec((RB, 1), lambda i: (i, 0)), _full_spec((1, dout))],
        out_specs=_row_spec(dout),
        out_shape=jax.ShapeDtypeStruct((n, dout), jnp.float32),
    )


def kernel(x, edge_index, percent, ricci_curvature, W1, b1, W2, b2, W3, b3,
           Wout, bout, bn1_w, bn1_b, bn2_w, bn2_b):
    n, din = x.shape
    e = edge_index.shape[1]
    h = W1.shape[1]
    dout = Wout.shape[1]

    ncht = e // CH
    tmax = ncht // NW + (1 if ncht % NW else 0)
    epad = (ncht + NW) * CH
    src = jnp.pad(edge_index[0], (0, epad - e))
    dst = jnp.pad(edge_index[1], (0, epad - e)).reshape(epad // CH, CH)
    zeros_h = jnp.zeros((CH, h), jnp.float32)

    b1r = b1.reshape(1, h)
    b2r = b2.reshape(1, h)
    b3r = b3.reshape(1, h)
    boutr = bout.reshape(1, dout)
    bn1wr = bn1_w.reshape(1, din)
    bn1br = bn1_b.reshape(1, din)
    bn2wr = bn2_w.reshape(1, 3 * h)
    bn2br = bn2_b.reshape(1, 3 * h)

    degp = _make_deg(n, e)(edge_index[1]).reshape(NW, _npad(n))
    dinv, z1 = _make_tc1(n, din, h)(x, bn1wr, bn1br, degp, W1)
    prop_h = _make_prop(n, e, h)
    s1 = prop_h(z1, src, dst, zeros_h)
    h1, z2 = _make_tc_mid(n, h)(s1, z1, dinv, b1r, W2)
    s2 = prop_h(z2, src, dst, zeros_h)
    h2, z3 = _make_tc_mid(n, h)(s2, z2, dinv, b2r, W3)
    s3 = prop_h(z3, src, dst, zeros_h)
    h3, z4 = _make_tc4(n, h, dout)(s3, z3, dinv, b3r, bn2wr, bn2br, Wout, h1, h2)
    s4 = prop_h(z4, src, dst, zeros_h)
    out = _make_tc5(n, h, dout)(s4, z4, dinv, boutr)
    return out, h1, h2, h3


# speedup vs baseline: 26.6679x; 1.0570x over previous
import functools

import jax
import jax.numpy as jnp
from jax import lax
from jax.experimental import pallas as pl
from jax.experimental.pallas import tpu as pltpu
from jax.experimental.pallas import tpu_sc as plsc

NC = 2
NS = 16
NW = NC * NS
CH = 96
DEGW = 8
RB = 1280
BN_C = 1.0 / (1.0 + 1e-5) ** 0.5


def _sc_mesh():
    return plsc.VectorSubcoreMesh(core_axis_name="c", subcore_axis_name="s")


def _npad(n):
    return ((n + 8 * NS - 1) // (8 * NS)) * (8 * NS)


@functools.lru_cache(maxsize=None)
def _make_deg(n, e):
    ept = e // NW
    npad = _npad(n)
    grows = npad // 128

    @functools.partial(
        pl.kernel,
        out_type=jax.ShapeDtypeStruct((NW, grows, 128), jnp.float32),
        mesh=_sc_mesh(),
        compiler_params=pltpu.CompilerParams(needs_layout_passes=False),
        scratch_types=[
            pltpu.VMEM((ept,), jnp.int32),
            pltpu.VMEM((grows, 128), jnp.float32),
        ],
    )
    def deg_kernel(dst_hbm, out_hbm, idx_v, deg_v):
        cid = lax.axis_index("c")
        sid = lax.axis_index("s")
        wid = sid * NC + cid

        def zero(i, carry):
            for t in range(8):
                deg_v[i, pl.ds(t * 16, 16)] = jnp.zeros((16,), jnp.float32)
            return carry

        lax.fori_loop(0, grows, zero, 0)
        pltpu.sync_copy(dst_hbm.at[pl.ds(pl.multiple_of(wid * ept, 8), ept)], idx_v)
        ones16 = jnp.ones((16,), jnp.float32)

        def body(j, carry):
            for t in range(5):
                idx = idx_v[pl.ds(j * 80 + t * 16, 16)]
                plsc.addupdate_scatter(deg_v, [idx >> 7, idx & 127], ones16)
            return carry

        lax.fori_loop(0, ept // 80, body, 0)
        pltpu.sync_copy(deg_v, out_hbm.at[wid])

    return deg_kernel


@functools.lru_cache(maxsize=None)
def _make_prop(n, e, d):
    ncht = -(-e // CH)
    tbase = ncht // NW
    rem = ncht % NW
    tmax = tbase + (1 if rem else 0)
    nslot = -(-tmax // 3) * 3
    npad = _npad(n)
    slab = npad // NS
    nwb = slab // CH
    wtail = slab - nwb * CH

    @functools.partial(
        pl.kernel,
        out_type=jax.ShapeDtypeStruct((NC, npad, d), jnp.float32),
        mesh=_sc_mesh(),
        scratch_types=[
            pltpu.VMEM((tmax * CH,), jnp.int32),
            pltpu.VMEM((CH,), jnp.int32),
            pltpu.VMEM((CH,), jnp.int32),
            pltpu.VMEM((CH,), jnp.int32),
            pltpu.VMEM((CH, d), jnp.float32),
            pltpu.VMEM((CH, d), jnp.float32),
            pltpu.VMEM((CH, d), jnp.float32),
            pltpu.VMEM_SHARED((npad, d), jnp.float32),
            pltpu.SemaphoreType.DMA,
            pltpu.SemaphoreType.DMA,
            pltpu.SemaphoreType.DMA,
            pltpu.SemaphoreType.DMA,
            pltpu.SemaphoreType.DMA,
            pltpu.SemaphoreType.DMA,
        ],
    )
    def prop_kernel(z_hbm, src_hbm, dst_hbm, zeros_hbm, out_hbm,
                    sidx_all, di0, di1, di2, row0, row1, row2, acc,
                    sg0, sg1, sg2, ss0, ss1, ss2):
        cid = lax.axis_index("c")
        sid = lax.axis_index("s")
        wid = sid * NC + cid
        cb = tbase * wid + jnp.minimum(wid, rem)
        tw = tbase + (wid < rem).astype(jnp.int32)

        rows = (row0, row1, row2)
        dis = (di0, di1, di2)
        sgs = (sg0, sg1, sg2)
        sss = (ss0, ss1, ss2)

        def gather(b, c):
            pltpu.async_copy(dst_hbm.at[cb + c], dis[b], sgs[b])
            pltpu.async_copy(z_hbm.at[sidx_all.at[pl.ds(c * CH, CH)]],
                             rows[b], sgs[b])

        def wait_gather(b):
            pltpu.make_async_copy(z_hbm.at[pl.ds(0, CH)], rows[b], sgs[b]).wait()
            pltpu.make_async_copy(dst_hbm.at[0], dis[b], sgs[b]).wait()

        def scatter(b):
            pltpu.async_copy(rows[b], acc.at[dis[b]], ss[b], add=True)

        def wait_scatter(b):
            pltpu.make_async_copy(rows[b], acc.at[dis[b]], sss[b]).wait()

        ss = sss

        ld_s = pltpu.async_copy(
            src_hbm.at[pl.ds(pl.multiple_of(cb * CH, 8), tmax * CH)],
            sidx_all, sg0)
        pltpu.sync_copy(zeros_hbm, row1)

        def zinit(k, carry):
            off = pl.multiple_of(sid * slab + k * CH, 8)
            pltpu.sync_copy(row1, acc.at[pl.ds(off, CH)])
            return carry

        lax.fori_loop(0, nwb, zinit, 0)
        if wtail:
            off = pl.multiple_of(sid * slab + nwb * CH, 8)
            pltpu.sync_copy(row1.at[pl.ds(0, wtail)], acc.at[pl.ds(off, wtail)])
        pltpu.make_async_copy(
            src_hbm.at[pl.ds(0, tmax * CH)], sidx_all, sg0).wait()
        plsc.subcore_barrier()

        gather(0, 0)
        gather(1, 1)

        def body(i, carry):
            j = 3 * i
            for t in range(3):
                s = j + t
                b = t

                @pl.when(s < tw)
                def _():
                    wait_gather(b)
                    scatter(b)

                bn = (t + 2) % 3

                @pl.when(s + 2 < tw)
                def _():
                    if t == 0:
                        @pl.when(i > 0)
                        def _():
                            wait_scatter(bn)
                    else:
                        wait_scatter(bn)
                    gather(bn, s + 2)

            return carry

        lax.fori_loop(0, nslot // 3, body, 0)
        wait_scatter(0)
        wait_scatter(1)
        wait_scatter(2)
        plsc.subcore_barrier()

        def wback(k, carry):
            off = pl.multiple_of(sid * slab + k * CH, 8)
            pltpu.sync_copy(acc.at[pl.ds(off, CH)], row0)
            pltpu.sync_copy(row0, out_hbm.at[cid, pl.ds(off, CH)])
            return carry

        lax.fori_loop(0, nwb, wback, 0)
        if wtail:
            off = pl.multiple_of(sid * slab + nwb * CH, 8)
            pltpu.sync_copy(acc.at[pl.ds(off, wtail)], row0.at[pl.ds(0, wtail)])
            pltpu.sync_copy(row0.at[pl.ds(0, wtail)],
                            out_hbm.at[cid, pl.ds(off, wtail)])


    return prop_kernel


def _row_spec(d):
    return pl.BlockSpec((RB, d), lambda i: (i, 0))


def _full_spec(shape):
    nd = len(shape)
    return pl.BlockSpec(shape, lambda i: (0,) * nd)


def _part_spec(d):
    return pl.BlockSpec((NC, RB, d), lambda i: (0, i, 0))


@functools.lru_cache(maxsize=None)
def _make_tc1(n, din, h):
    G = RB // 128

    def body(x_ref, bnw_ref, bnb_ref, degp_ref, w_ref, dinv_ref, z_ref):
        dsum = jnp.sum(degp_ref[...], axis=0, keepdims=True) + 1.0
        io = lax.broadcasted_iota(jnp.int32, (128, 128), 0)
        ic = lax.broadcasted_iota(jnp.int32, (128, 128), 1)
        eye = (io == ic).astype(jnp.float32)
        onesc = jnp.ones((128, 1), jnp.float32)
        cols = [jnp.dot(eye * dsum[:, g * 128:(g + 1) * 128], onesc,
                        preferred_element_type=jnp.float32) for g in range(G)]
        deg = jnp.concatenate(cols, axis=0)
        dinv = lax.rsqrt(jnp.maximum(deg, 1.0))
        dinv_ref[...] = dinv
        xb = x_ref[...] * (BN_C * bnw_ref[...]) + bnb_ref[...]
        z = jnp.dot(xb, w_ref[...], preferred_element_type=jnp.float32)
        z_ref[...] = z * dinv

    return pl.pallas_call(
        body,
        grid=(pl.cdiv(n, RB),),
        in_specs=[_row_spec(din), _full_spec((1, din)), _full_spec((1, din)),
                  pl.BlockSpec((NW, RB), lambda i: (0, i)),
                  _full_spec((din, h))],
        out_specs=[pl.BlockSpec((RB, 1), lambda i: (i, 0)), _row_spec(h)],
        out_shape=[jax.ShapeDtypeStruct((n, 1), jnp.float32),
                   jax.ShapeDtypeStruct((n, h), jnp.float32)],
    )


@functools.lru_cache(maxsize=None)
def _make_tc_mid(n, h):
    def body(sp_ref, z_ref, dinv_ref, b_ref, w_ref, h_ref, znext_ref):
        dinv = dinv_ref[...]
        g = (sp_ref[0] + sp_ref[1] + z_ref[...]) * dinv + b_ref[...]
        hv = jnp.maximum(g, 0.0)
        h_ref[...] = hv
        znext = jnp.dot(hv, w_ref[...], preferred_element_type=jnp.float32)
        znext_ref[...] = znext * dinv

    return pl.pallas_call(
        body,
        grid=(pl.cdiv(n, RB),),
        in_specs=[_part_spec(h), _row_spec(h), pl.BlockSpec((RB, 1), lambda i: (i, 0)),
                  _full_spec((1, h)), _full_spec((h, h))],
        out_specs=[_row_spec(h), _row_spec(h)],
        out_shape=[jax.ShapeDtypeStruct((n, h), jnp.float32),
                   jax.ShapeDtypeStruct((n, h), jnp.float32)],
    )


@functools.lru_cache(maxsize=None)
def _make_tc4(n, h, dout):
    def body(sp_ref, z_ref, dinv_ref, b_ref, bn2w_ref, bn2b_ref, wout_ref,
             h1_ref, h2_ref, h3_ref, z4_ref):
        dinv = dinv_ref[...]
        g = (sp_ref[0] + sp_ref[1] + z_ref[...]) * dinv + b_ref[...]
        h3 = jnp.maximum(g, 0.0)
        h3_ref[...] = h3
        z4 = jnp.zeros((RB, dout), jnp.float32)
        for k, hk in enumerate((h1_ref[...], h2_ref[...], h3)):
            xk = hk * (BN_C * bn2w_ref[:, k * h:(k + 1) * h]) + bn2b_ref[:, k * h:(k + 1) * h]
            z4 = z4 + jnp.dot(xk, wout_ref[k * h:(k + 1) * h, :],
                              preferred_element_type=jnp.float32)
        z4_ref[...] = jnp.concatenate(
            [z4 * dinv, jnp.zeros((RB, h - dout), jnp.float32)], axis=1)

    return pl.pallas_call(
        body,
        grid=(pl.cdiv(n, RB),),
        in_specs=[_part_spec(h), _row_spec(h), pl.BlockSpec((RB, 1), lambda i: (i, 0)),
                  _full_spec((1, h)), _full_spec((1, 3 * h)), _full_spec((1, 3 * h)),
                  _full_spec((3 * h, dout)), _row_spec(h), _row_spec(h)],
        out_specs=[_row_spec(h), _row_spec(h)],
        out_shape=[jax.ShapeDtypeStruct((n, h), jnp.float32),
                   jax.ShapeDtypeStruct((n, h), jnp.float32)],
    )


@functools.lru_cache(maxsize=None)
def _make_tc5(n, h, dout):
    def body(sp_ref, z_ref, dinv_ref, b_ref, out_ref):
        s = (sp_ref[0] + sp_ref[1] + z_ref[...]) * dinv_ref[...]
        out_ref[...] = s[:, :dout] + b_ref[...]

    return pl.pallas_call(
        body,
        grid=(pl.cdiv(n, RB),),
        in_specs=[_part_spec(h), _row_spec(h),
                  pl.BlockSpec((RB, 1), lambda i: (i, 0)), _full_spec((1, dout))],
        out_specs=_row_spec(dout),
        out_shape=jax.ShapeDtypeStruct((n, dout), jnp.float32),
    )


def kernel(x, edge_index, percent, ricci_curvature, W1, b1, W2, b2, W3, b3,
           Wout, bout, bn1_w, bn1_b, bn2_w, bn2_b):
    n, din = x.shape
    e = edge_index.shape[1]
    h = W1.shape[1]
    dout = Wout.shape[1]

    ncht = -(-e // CH)
    epad = (ncht + NW) * CH
    src = jnp.pad(edge_index[0], (0, epad - e))
    dst = jnp.pad(edge_index[1], (0, epad - e),
                  constant_values=_npad(n) - 1).reshape(epad // CH, CH)
    zeros_h = jnp.zeros((CH, h), jnp.float32)

    b1r = b1.reshape(1, h)
    b2r = b2.reshape(1, h)
    b3r = b3.reshape(1, h)
    boutr = bout.reshape(1, dout)
    bn1wr = bn1_w.reshape(1, din)
    bn1br = bn1_b.reshape(1, din)
    bn2wr = bn2_w.reshape(1, 3 * h)
    bn2br = bn2_b.reshape(1, 3 * h)

    degp = _make_deg(n, e)(edge_index[1]).reshape(NW, _npad(n))
    dinv, z1 = _make_tc1(n, din, h)(x, bn1wr, bn1br, degp, W1)
    prop_h = _make_prop(n, e, h)
    s1 = prop_h(z1, src, dst, zeros_h)
    h1, z2 = _make_tc_mid(n, h)(s1, z1, dinv, b1r, W2)
    s2 = prop_h(z2, src, dst, zeros_h)
    h2, z3 = _make_tc_mid(n, h)(s2, z2, dinv, b2r, W3)
    s3 = prop_h(z3, src, dst, zeros_h)
    h3, z4 = _make_tc4(n, h, dout)(s3, z3, dinv, b3r, bn2wr, bn2br, Wout, h1, h2)
    s4 = prop_h(z4, src, dst, zeros_h)
    out = _make_tc5(n, h, dout)(s4, z4, dinv, boutr)
    return out, h1, h2, h3


# speedup vs baseline: 26.7115x; 1.0016x over previous
import functools

import jax
import jax.numpy as jnp
from jax import lax
from jax.experimental import pallas as pl
from jax.experimental.pallas import tpu as pltpu
from jax.experimental.pallas import tpu_sc as plsc

NC = 2
NS = 16
NW = NC * NS
CH = 96
DEGW = 8
RB = 1280
BN_C = 1.0 / (1.0 + 1e-5) ** 0.5


def _sc_mesh():
    return plsc.VectorSubcoreMesh(core_axis_name="c", subcore_axis_name="s")


def _npad(n):
    return ((n + 8 * NS - 1) // (8 * NS)) * (8 * NS)


@functools.lru_cache(maxsize=None)
def _make_deg(n, e):
    ept = e // NW
    npad = _npad(n)
    grows = npad // 128

    @functools.partial(
        pl.kernel,
        out_type=jax.ShapeDtypeStruct((NW, grows, 128), jnp.float32),
        mesh=_sc_mesh(),
        compiler_params=pltpu.CompilerParams(needs_layout_passes=False),
        scratch_types=[
            pltpu.VMEM((ept,), jnp.int32),
            pltpu.VMEM((grows, 128), jnp.float32),
        ],
    )
    def deg_kernel(dst_hbm, out_hbm, idx_v, deg_v):
        cid = lax.axis_index("c")
        sid = lax.axis_index("s")
        wid = sid * NC + cid

        def zero(i, carry):
            for t in range(8):
                deg_v[i, pl.ds(t * 16, 16)] = jnp.zeros((16,), jnp.float32)
            return carry

        lax.fori_loop(0, grows, zero, 0)
        pltpu.sync_copy(dst_hbm.at[pl.ds(pl.multiple_of(wid * ept, 8), ept)], idx_v)
        ones16 = jnp.ones((16,), jnp.float32)

        def body(j, carry):
            for t in range(5):
                idx = idx_v[pl.ds(j * 80 + t * 16, 16)]
                plsc.addupdate_scatter(deg_v, [idx >> 7, idx & 127], ones16)
            return carry

        lax.fori_loop(0, ept // 80, body, 0)
        pltpu.sync_copy(deg_v, out_hbm.at[wid])

    return deg_kernel


@functools.lru_cache(maxsize=None)
def _make_prop(n, e, d):
    ncht = -(-e // CH)
    tbase = ncht // NW
    rem = ncht % NW
    tmax = tbase + (1 if rem else 0)
    nslot = -(-tmax // 3) * 3
    npad = _npad(n)
    slab = npad // NS
    nwb = slab // CH
    wtail = slab - nwb * CH

    @functools.partial(
        pl.kernel,
        out_type=jax.ShapeDtypeStruct((NC, npad, d), jnp.float32),
        mesh=_sc_mesh(),
        scratch_types=[
            pltpu.VMEM((tmax * CH,), jnp.int32),
            pltpu.VMEM((CH,), jnp.int32),
            pltpu.VMEM((CH,), jnp.int32),
            pltpu.VMEM((CH,), jnp.int32),
            pltpu.VMEM((CH, d), jnp.float32),
            pltpu.VMEM((CH, d), jnp.float32),
            pltpu.VMEM((CH, d), jnp.float32),
            pltpu.VMEM_SHARED((npad, d), jnp.float32),
            pltpu.SemaphoreType.DMA,
            pltpu.SemaphoreType.DMA,
            pltpu.SemaphoreType.DMA,
            pltpu.SemaphoreType.DMA,
            pltpu.SemaphoreType.DMA,
            pltpu.SemaphoreType.DMA,
        ],
    )
    def prop_kernel(z_hbm, src_hbm, dst_hbm, zeros_hbm, out_hbm,
                    sidx_all, di0, di1, di2, row0, row1, row2, acc,
                    sg0, sg1, sg2, ss0, ss1, ss2):
        cid = lax.axis_index("c")
        sid = lax.axis_index("s")
        wid = sid * NC + cid
        cb = tbase * wid + jnp.minimum(wid, rem)
        tw = tbase + (wid < rem).astype(jnp.int32)

        rows = (row0, row1, row2)
        dis = (di0, di1, di2)
        sgs = (sg0, sg1, sg2)
        sss = (ss0, ss1, ss2)

        def gather(b, c):
            pltpu.async_copy(dst_hbm.at[cb + c], dis[b], sgs[b])
            pltpu.async_copy(z_hbm.at[sidx_all.at[pl.ds(c * CH, CH)]],
                             rows[b], sgs[b])

        def wait_gather(b):
            pltpu.make_async_copy(z_hbm.at[pl.ds(0, CH)], rows[b], sgs[b]).wait()
            pltpu.make_async_copy(dst_hbm.at[0], dis[b], sgs[b]).wait()

        def scatter(b):
            pltpu.async_copy(rows[b], acc.at[dis[b]], sss[b], add=True)

        def wait_scatter(b):
            pltpu.make_async_copy(rows[b], acc.at[dis[b]], sss[b]).wait()

        ld_s = pltpu.async_copy(
            src_hbm.at[pl.ds(pl.multiple_of(cb * CH, 8), tmax * CH)],
            sidx_all, sg0)
        pltpu.sync_copy(zeros_hbm, row1)

        def zinit(k, carry):
            off = pl.multiple_of(sid * slab + k * CH, 8)
            pltpu.sync_copy(row1, acc.at[pl.ds(off, CH)])
            return carry

        lax.fori_loop(0, nwb, zinit, 0)
        if wtail:
            off = pl.multiple_of(sid * slab + nwb * CH, 8)
            pltpu.sync_copy(row1.at[pl.ds(0, wtail)], acc.at[pl.ds(off, wtail)])
        pltpu.make_async_copy(
            src_hbm.at[pl.ds(0, tmax * CH)], sidx_all, sg0).wait()
        plsc.subcore_barrier()

        gather(0, 0)
        gather(1, 1)

        def body(i, carry):
            j = 3 * i
            for t in range(3):
                s = j + t
                b = t

                @pl.when(s < tw)
                def _():
                    wait_gather(b)
                    scatter(b)

                bn = (t + 2) % 3

                @pl.when(s + 2 < tw)
                def _():
                    if t == 0:
                        @pl.when(i > 0)
                        def _():
                            wait_scatter(bn)
                    else:
                        wait_scatter(bn)
                    gather(bn, s + 2)

            return carry

        lax.fori_loop(0, nslot // 3, body, 0)
        wait_scatter(0)
        wait_scatter(1)
        wait_scatter(2)
        plsc.subcore_barrier()

        def wback(k, carry):
            off = pl.multiple_of(sid * slab + k * CH, 8)
            pltpu.sync_copy(acc.at[pl.ds(off, CH)], row0)
            pltpu.sync_copy(row0, out_hbm.at[cid, pl.ds(off, CH)])
            return carry

        lax.fori_loop(0, nwb, wback, 0)
        if wtail:
            off = pl.multiple_of(sid * slab + nwb * CH, 8)
            pltpu.sync_copy(acc.at[pl.ds(off, wtail)], row0.at[pl.ds(0, wtail)])
            pltpu.sync_copy(row0.at[pl.ds(0, wtail)],
                            out_hbm.at[cid, pl.ds(off, wtail)])


    return prop_kernel


def _row_spec(d):
    return pl.BlockSpec((RB, d), lambda i: (i, 0))


def _full_spec(shape):
    nd = len(shape)
    return pl.BlockSpec(shape, lambda i: (0,) * nd)


def _part_spec(d):
    return pl.BlockSpec((NC, RB, d), lambda i: (0, i, 0))


@functools.lru_cache(maxsize=None)
def _make_tc1(n, din, h):
    G = RB // 128

    def body(x_ref, bnw_ref, bnb_ref, degp_ref, w_ref, dinv_ref, z_ref):
        dsum = jnp.sum(degp_ref[...], axis=0, keepdims=True) + 1.0
        io = lax.broadcasted_iota(jnp.int32, (128, 128), 0)
        ic = lax.broadcasted_iota(jnp.int32, (128, 128), 1)
        eye = (io == ic).astype(jnp.float32)
        onesc = jnp.ones((128, 1), jnp.float32)
        cols = [jnp.dot(eye * dsum[:, g * 128:(g + 1) * 128], onesc,
                        preferred_element_type=jnp.float32) for g in range(G)]
        deg = jnp.concatenate(cols, axis=0)
        dinv = lax.rsqrt(jnp.maximum(deg, 1.0))
        dinv_ref[...] = dinv
        xb = x_ref[...] * (BN_C * bnw_ref[...]) + bnb_ref[...]
        z = jnp.dot(xb, w_ref[...], preferred_element_type=jnp.float32)
        z_ref[...] = z * dinv

    return pl.pallas_call(
        body,
        grid=(pl.cdiv(n, RB),),
        in_specs=[_row_spec(din), _full_spec((1, din)), _full_spec((1, din)),
                  pl.BlockSpec((NW, RB), lambda i: (0, i)),
                  _full_spec((din, h))],
        out_specs=[pl.BlockSpec((RB, 1), lambda i: (i, 0)), _row_spec(h)],
        out_shape=[jax.ShapeDtypeStruct((n, 1), jnp.float32),
                   jax.ShapeDtypeStruct((n, h), jnp.float32)],
    )


@functools.lru_cache(maxsize=None)
def _make_tc_mid(n, h):
    def body(sp_ref, z_ref, dinv_ref, b_ref, w_ref, h_ref, znext_ref):
        dinv = dinv_ref[...]
        g = (sp_ref[0] + sp_ref[1] + z_ref[...]) * dinv + b_ref[...]
        hv = jnp.maximum(g, 0.0)
        h_ref[...] = hv
        znext = jnp.dot(hv, w_ref[...], preferred_element_type=jnp.float32)
        znext_ref[...] = znext * dinv

    return pl.pallas_call(
        body,
        grid=(pl.cdiv(n, RB),),
        in_specs=[_part_spec(h), _row_spec(h), pl.BlockSpec((RB, 1), lambda i: (i, 0)),
                  _full_spec((1, h)), _full_spec((h, h))],
        out_specs=[_row_spec(h), _row_spec(h)],
        out_shape=[jax.ShapeDtypeStruct((n, h), jnp.float32),
                   jax.ShapeDtypeStruct((n, h), jnp.float32)],
    )


@functools.lru_cache(maxsize=None)
def _make_tc4(n, h, dout):
    def body(sp_ref, z_ref, dinv_ref, b_ref, bn2w_ref, bn2b_ref, wout_ref,
             h1_ref, h2_ref, h3_ref, z4_ref):
        dinv = dinv_ref[...]
        g = (sp_ref[0] + sp_ref[1] + z_ref[...]) * dinv + b_ref[...]
        h3 = jnp.maximum(g, 0.0)
        h3_ref[...] = h3
        z4 = jnp.zeros((RB, dout), jnp.float32)
        for k, hk in enumerate((h1_ref[...], h2_ref[...], h3)):
            xk = hk * (BN_C * bn2w_ref[:, k * h:(k + 1) * h]) + bn2b_ref[:, k * h:(k + 1) * h]
            z4 = z4 + jnp.dot(xk, wout_ref[k * h:(k + 1) * h, :],
                              preferred_element_type=jnp.float32)
        z4_ref[...] = jnp.concatenate(
            [z4 * dinv, jnp.zeros((RB, h - dout), jnp.float32)], axis=1)

    return pl.pallas_call(
        body,
        grid=(pl.cdiv(n, RB),),
        in_specs=[_part_spec(h), _row_spec(h), pl.BlockSpec((RB, 1), lambda i: (i, 0)),
                  _full_spec((1, h)), _full_spec((1, 3 * h)), _full_spec((1, 3 * h)),
                  _full_spec((3 * h, dout)), _row_spec(h), _row_spec(h)],
        out_specs=[_row_spec(h), _row_spec(h)],
        out_shape=[jax.ShapeDtypeStruct((n, h), jnp.float32),
                   jax.ShapeDtypeStruct((n, h), jnp.float32)],
    )


@functools.lru_cache(maxsize=None)
def _make_tc5(n, h, dout):
    def body(sp_ref, z_ref, dinv_ref, b_ref, out_ref):
        s = (sp_ref[0] + sp_ref[1] + z_ref[...]) * dinv_ref[...]
        out_ref[...] = s[:, :dout] + b_ref[...]

    return pl.pallas_call(
        body,
        grid=(pl.cdiv(n, RB),),
        in_specs=[_part_spec(h), _row_spec(h),
                  pl.BlockSpec((RB, 1), lambda i: (i, 0)), _full_spec((1, dout))],
        out_specs=_row_spec(dout),
        out_shape=jax.ShapeDtypeStruct((n, dout), jnp.float32),
    )


def kernel(x, edge_index, percent, ricci_curvature, W1, b1, W2, b2, W3, b3,
           Wout, bout, bn1_w, bn1_b, bn2_w, bn2_b):
    n, din = x.shape
    e = edge_index.shape[1]
    h = W1.shape[1]
    dout = Wout.shape[1]

    ncht = -(-e // CH)
    epad = (ncht + NW) * CH
    src = jnp.pad(edge_index[0], (0, epad - e))
    dst = jnp.pad(edge_index[1], (0, epad - e),
                  constant_values=_npad(n) - 1).reshape(epad // CH, CH)
    zeros_h = jnp.zeros((CH, h), jnp.float32)

    b1r = b1.reshape(1, h)
    b2r = b2.reshape(1, h)
    b3r = b3.reshape(1, h)
    boutr = bout.reshape(1, dout)
    bn1wr = bn1_w.reshape(1, din)
    bn1br = bn1_b.reshape(1, din)
    bn2wr = bn2_w.reshape(1, 3 * h)
    bn2br = bn2_b.reshape(1, 3 * h)

    degp = _make_deg(n, e)(edge_index[1]).reshape(NW, _npad(n))
    dinv, z1 = _make_tc1(n, din, h)(x, bn1wr, bn1br, degp, W1)
    prop_h = _make_prop(n, e, h)
    s1 = prop_h(z1, src, dst, zeros_h)
    h1, z2 = _make_tc_mid(n, h)(s1, z1, dinv, b1r, W2)
    s2 = prop_h(z2, src, dst, zeros_h)
    h2, z3 = _make_tc_mid(n, h)(s2, z2, dinv, b2r, W3)
    s3 = prop_h(z3, src, dst, zeros_h)
    h3, z4 = _make_tc4(n, h, dout)(s3, z3, dinv, b3r, bn2wr, bn2br, Wout, h1, h2)
    s4 = prop_h(z4, src, dst, zeros_h)
    out = _make_tc5(n, h, dout)(s4, z4, dinv, boutr)
    return out, h1, h2, h3
